# baseline (speedup 1.0000x reference)
import jax
import jax.numpy as jnp
from jax.experimental import pallas as pl
from jax.experimental.pallas import tpu as pltpu

_DIM = 96
_E = 8
_TOPK = 2
_R = 8
_ALPHA = 8
_B, _H, _W = 64, 56, 56
_HW = _H * _W
_NROWS = 64
_IPB = 8
_NBLK = _B // _IPB


def _moe_stream_kernel(
    x_ref, rw_ref, rb_ref, wd_ref, wu_ref, out_ref, sum_ref
):
    j = pl.program_id(0)
    nblk = pl.num_programs(0)
    img0 = ((j + 1) % nblk) * _IPB

    blk = x_ref[:, :, :]
    out_ref[:, :, :] = blk
    sum_ref[pl.ds(img0, _IPB), :] = jnp.sum(blk, axis=1)

    @pl.when(j == nblk - 1)
    def _finalize():
        x_mean = sum_ref[:, :] * (1.0 / _HW)
        logits = (
            jnp.dot(x_mean, rw_ref[:, :], preferred_element_type=jnp.float32)
            + rb_ref[0, :]
        )
        gate = jax.nn.softmax(logits, axis=-1)

        iota = jax.lax.broadcasted_iota(jnp.int32, gate.shape, 1)
        m1 = jnp.max(gate, axis=-1, keepdims=True)
        i1 = jnp.min(jnp.where(gate == m1, iota, _E), axis=-1, keepdims=True)
        hot1 = (iota == i1).astype(jnp.float32)
        gate2 = gate - hot1 * 2.0
        m2 = jnp.max(gate2, axis=-1, keepdims=True)
        i2 = jnp.min(jnp.where(gate2 == m2, iota, _E), axis=-1, keepdims=True)
        hot2 = (iota == i2).astype(jnp.float32)
        denom = m1 + m2 + 1e-6
        wt = (hot1 * m1 + hot2 * m2) / denom

        x_rows = blk[0, 0:_NROWS, :]
        scaling = float(_ALPHA) / float(_R)
        moe = jnp.zeros((_NROWS, _DIM), dtype=jnp.float32)
        for i in range(_E):
            h = jnp.dot(x_rows, wd_ref[i, :, :], preferred_element_type=jnp.float32)
            h = 0.5 * h * (1.0 + jax.lax.erf(h * (2.0 ** -0.5)))
            h = jnp.dot(h, wu_ref[i, :, :], preferred_element_type=jnp.float32)
            moe = moe + h * wt[:, i : i + 1]
        out_ref[0, 0:_NROWS, :] = x_rows + moe * scaling


def kernel(x, router_w, router_b, w_down, w_up):
    x3 = x.reshape(_B, _HW, _DIM)
    rb2 = router_b.reshape(1, _E)
    out = pl.pallas_call(
        _moe_stream_kernel,
        grid=(_NBLK,),
        in_specs=[
            pl.BlockSpec((_IPB, _HW, _DIM), lambda j: ((j + 1) % _NBLK, 0, 0)),
            pl.BlockSpec((_DIM, _E), lambda j: (0, 0)),
            pl.BlockSpec((1, _E), lambda j: (0, 0)),
            pl.BlockSpec((_E, _DIM, _R), lambda j: (0, 0, 0)),
            pl.BlockSpec((_E, _R, _DIM), lambda j: (0, 0, 0)),
        ],
        out_specs=pl.BlockSpec((_IPB, _HW, _DIM), lambda j: ((j + 1) % _NBLK, 0, 0)),
        out_shape=jax.ShapeDtypeStruct((_B, _HW, _DIM), x.dtype),
        scratch_shapes=[
            pltpu.VMEM((_B, _DIM), jnp.float32),
        ],
    )(x3, router_w, rb2, w_down, w_up)
    return out.reshape(x.shape)
